# Initial kernel scaffold; baseline (speedup 1.0000x reference)
import jax
import jax.numpy as jnp
from jax.experimental import pallas as pl


def kernel(position_matrix, channel_matrix, n_select_0, n_select_1, n_select_2, W1_0, b1_0, W1_1, b1_1, Wres1, bres1, ln1_g, ln1_b, W2_0, b2_0, W2_1, b2_1, Wres2, bres2, ln2_g, ln2_b):
    raise NotImplementedError("write your pallas kernel here")


# speedup vs baseline: 1.0007x; 1.0007x over previous
import jax
import jax.numpy as jnp
from jax.experimental import pallas as pl

K = 16


def _dc(pos, feat, n_out_static, Ws, bs, Wres, bres, res_in):
    Bn = pos.shape[0]
    new_pos = pos[:, :n_out_static]
    d = jnp.sum((new_pos[:, :, None, :] - pos[:, None, :, :]) ** 2, axis=-1)
    _, idx = jax.lax.top_k(-d, K)
    bidx = jnp.arange(Bn)[:, None, None]
    grouped_feat = feat[bidx, idx]
    grouped_pos = pos[bidx, idx]
    rel = grouped_pos - new_pos[:, :, None, :]
    x = jnp.concatenate([grouped_feat, rel], axis=-1)
    for W, b in zip(Ws, bs):
        x = jax.nn.relu(x @ W + b)
    out = jnp.max(x, axis=2)
    res = feat[:, :n_out_static] @ Wres + bres
    if res_in is not None:
        res = res + res_in[:, :n_out_static]
    return new_pos, out, res


def _ln(x, g, b):
    mu = jnp.mean(x, axis=-1, keepdims=True)
    var = jnp.var(x, axis=-1, keepdims=True)
    return (x - mu) / jnp.sqrt(var + 1e-5) * g + b


def kernel(position_matrix, channel_matrix, n_select_0, n_select_1, n_select_2, W1_0, b1_0, W1_1, b1_1, Wres1, bres1, ln1_g, ln1_b, W2_0, b2_0, W2_1, b2_1, Wres2, bres2, ln2_g, ln2_b):
    n0 = position_matrix.shape[1]
    n1 = n0 // 2
    n2 = n0 // 4
    pos = position_matrix
    feat = channel_matrix
    out_pos, out_ch, res_ch = _dc(pos, feat, n1, (W1_0, W1_1), (b1_0, b1_1), Wres1, bres1, None)
    out_ch = jax.nn.silu(_ln(out_ch, ln1_g, ln1_b))
    out_pos, out_ch, res_ch = _dc(out_pos, out_ch, n2, (W2_0, W2_1), (b2_0, b2_1), Wres2, bres2, res_ch)
    out_ch = jax.nn.silu(_ln(out_ch, ln2_g, ln2_b))
    out_ch = out_ch + res_ch
    return (out_pos, out_ch)


# speedup vs baseline: 1.5677x; 1.5666x over previous
import functools
import jax
import jax.numpy as jnp
from jax import lax
from jax.experimental import pallas as pl
from jax.experimental.pallas import tpu as pltpu

B = 4
N0 = 4096
N1 = 2048
N2 = 1024
C = 128
K = 16


def _d1_body(pos8_ref, nposT_ref, f_ref, w10_ref, d_ref, z_ref):
    p = pos8_ref[0]
    nT = nposT_ref[0]
    g = jnp.dot(p, nT, preferred_element_type=jnp.float32)
    rs = jnp.sum(p * p, axis=1, keepdims=True)
    cs = jnp.sum(nT * nT, axis=0, keepdims=True)
    d_ref[0] = rs + cs - 2.0 * g
    z_ref[0] = jnp.dot(f_ref[0], w10_ref[...], preferred_element_type=jnp.float32)


def _call_d1(pos8, nposT, f_pad, w10p):
    return pl.pallas_call(
        _d1_body,
        grid=(B, N0 // 512),
        in_specs=[
            pl.BlockSpec((1, 512, 8), lambda b, j: (b, j, 0)),
            pl.BlockSpec((1, 8, N1), lambda b, j: (b, 0, 0)),
            pl.BlockSpec((1, 512, 136), lambda b, j: (b, j, 0)),
            pl.BlockSpec((136, C), lambda b, j: (0, 0)),
        ],
        out_specs=[
            pl.BlockSpec((1, 512, N1), lambda b, j: (b, j, 0)),
            pl.BlockSpec((1, 512, C), lambda b, j: (b, j, 0)),
        ],
        out_shape=[
            jax.ShapeDtypeStruct((B, N0, N1), jnp.float32),
            jax.ShapeDtypeStruct((B, N0, C), jnp.float32),
        ],
    )(pos8, nposT, f_pad, w10p)


def _d2_body(pos8_ref, nposT_ref, feat1_ref, wres1_ref, bres1_ref, w1p_ref,
             b10_ref, w2p_ref, b20_ref, d_ref, c1_ref, res1_ref, np2_ref, c2_ref):
    jb = pl.program_id(1)
    p = pos8_ref[0]
    nT = nposT_ref[0]
    g = jnp.dot(p, nT, preferred_element_type=jnp.float32)
    rs = jnp.sum(p * p, axis=1, keepdims=True)
    cs = jnp.sum(nT * nT, axis=0, keepdims=True)
    d_ref[0] = rs + cs - 2.0 * g
    c1_ref[0] = b10_ref[...] - jnp.dot(p, w1p_ref[...], preferred_element_type=jnp.float32)
    res1_ref[0] = jnp.dot(feat1_ref[0], wres1_ref[...], preferred_element_type=jnp.float32) + bres1_ref[...]
    np2 = jnp.dot(p, w2p_ref[...], preferred_element_type=jnp.float32)
    np2_ref[0] = np2

    @pl.when(jb == 0)
    def _():
        c2_ref[0] = b20_ref[...] - np2


def _call_d2(pos2_8, npos2T, feat1, wres1, bres1, w1p8, b10, w2p8, b20):
    return pl.pallas_call(
        _d2_body,
        grid=(B, 2),
        in_specs=[
            pl.BlockSpec((1, 1024, 8), lambda b, j: (b, j, 0)),
            pl.BlockSpec((1, 8, N2), lambda b, j: (b, 0, 0)),
            pl.BlockSpec((1, 1024, C), lambda b, j: (b, j, 0)),
            pl.BlockSpec((C, C), lambda b, j: (0, 0)),
            pl.BlockSpec((1, C), lambda b, j: (0, 0)),
            pl.BlockSpec((8, C), lambda b, j: (0, 0)),
            pl.BlockSpec((1, C), lambda b, j: (0, 0)),
            pl.BlockSpec((8, C), lambda b, j: (0, 0)),
            pl.BlockSpec((1, C), lambda b, j: (0, 0)),
        ],
        out_specs=[
            pl.BlockSpec((1, 1024, N2), lambda b, j: (b, j, 0)),
            pl.BlockSpec((1, 1024, C), lambda b, j: (b, j, 0)),
            pl.BlockSpec((1, 1024, C), lambda b, j: (b, j, 0)),
            pl.BlockSpec((1, 1024, C), lambda b, j: (b, j, 0)),
            pl.BlockSpec((1, 1024, C), lambda b, j: (b, 0, 0)),
        ],
        out_shape=[
            jax.ShapeDtypeStruct((B, N1, N2), jnp.float32),
            jax.ShapeDtypeStruct((B, N1, C), jnp.float32),
            jax.ShapeDtypeStruct((B, N1, C), jnp.float32),
            jax.ShapeDtypeStruct((B, N1, C), jnp.float32),
            jax.ShapeDtypeStruct((B, N2, C), jnp.float32),
        ],
    )(pos2_8, npos2T, feat1, wres1, bres1, w1p8, b10, w2p8, b20)


def _mlp_body(g_ref, c_ref, w_ref, b_ref, lng_ref, lnb_ref, out_ref):
    cb = c_ref[0]
    w = w_ref[...]
    bb = b_ref[...]
    acc = jnp.zeros(out_ref.shape[1:], jnp.float32)
    for k in range(K):
        hk = jnp.maximum(g_ref[0][:, k * C:(k + 1) * C] + cb, 0.0)
        acc = jnp.maximum(acc, jnp.maximum(jnp.dot(hk, w, preferred_element_type=jnp.float32) + bb, 0.0))
    mu = jnp.mean(acc, axis=1, keepdims=True)
    xc = acc - mu
    var = jnp.mean(xc * xc, axis=1, keepdims=True)
    ln = xc * lax.rsqrt(var + 1e-5) * lng_ref[...] + lnb_ref[...]
    out_ref[0] = ln * jax.nn.sigmoid(ln)


def _call_mlp(g, c, w, b, lng, lnb, n_rows, blk):
    return pl.pallas_call(
        _mlp_body,
        grid=(B, n_rows // blk),
        in_specs=[
            pl.BlockSpec((1, blk, K * C), lambda b_, i: (b_, i, 0)),
            pl.BlockSpec((1, blk, C), lambda b_, i: (b_, i, 0)),
            pl.BlockSpec((C, C), lambda b_, i: (0, 0)),
            pl.BlockSpec((1, C), lambda b_, i: (0, 0)),
            pl.BlockSpec((1, C), lambda b_, i: (0, 0)),
            pl.BlockSpec((1, C), lambda b_, i: (0, 0)),
        ],
        out_specs=pl.BlockSpec((1, blk, C), lambda b_, i: (b_, i, 0)),
        out_shape=jax.ShapeDtypeStruct((B, n_rows, C), jnp.float32),
    )(g, c, w, b, lng, lnb)


def _b2_body(och_ref, np2_ref, w2f_ref, wres2_ref, bres2_ref, res1_ref,
             z2_ref, res2_ref):
    och = och_ref[0]
    z2_ref[0] = jnp.dot(och, w2f_ref[...], preferred_element_type=jnp.float32) + np2_ref[0]
    res2_ref[0] = (jnp.dot(och[:N2], wres2_ref[...], preferred_element_type=jnp.float32)
                   + bres2_ref[...] + res1_ref[0])


def _call_b2(och, np2, w2f, wres2, bres2, res1):
    return pl.pallas_call(
        _b2_body,
        grid=(B,),
        in_specs=[
            pl.BlockSpec((1, N1, C), lambda b: (b, 0, 0)),
            pl.BlockSpec((1, N1, C), lambda b: (b, 0, 0)),
            pl.BlockSpec((C, C), lambda b: (0, 0)),
            pl.BlockSpec((C, C), lambda b: (0, 0)),
            pl.BlockSpec((1, C), lambda b: (0, 0)),
            pl.BlockSpec((1, N2, C), lambda b: (b, 0, 0)),
        ],
        out_specs=[
            pl.BlockSpec((1, N1, C), lambda b: (b, 0, 0)),
            pl.BlockSpec((1, N2, C), lambda b: (b, 0, 0)),
        ],
        out_shape=[
            jax.ShapeDtypeStruct((B, N1, C), jnp.float32),
            jax.ShapeDtypeStruct((B, N2, C), jnp.float32),
        ],
    )(och, np2, w2f, wres2, bres2, res1)


def _c_body(g_ref, c_ref, w_ref, b_ref, lng_ref, lnb_ref, res2_ref, out_ref):
    cb = c_ref[0]
    w = w_ref[...]
    bb = b_ref[...]
    acc = jnp.zeros(out_ref.shape[1:], jnp.float32)
    for k in range(K):
        hk = jnp.maximum(g_ref[0][:, k * C:(k + 1) * C] + cb, 0.0)
        acc = jnp.maximum(acc, jnp.maximum(jnp.dot(hk, w, preferred_element_type=jnp.float32) + bb, 0.0))
    mu = jnp.mean(acc, axis=1, keepdims=True)
    xc = acc - mu
    var = jnp.mean(xc * xc, axis=1, keepdims=True)
    ln = xc * lax.rsqrt(var + 1e-5) * lng_ref[...] + lnb_ref[...]
    out_ref[0] = ln * jax.nn.sigmoid(ln) + res2_ref[0]


def _call_c(g2, c2, w21, b21, lng, lnb, res2):
    return pl.pallas_call(
        _c_body,
        grid=(B, 2),
        in_specs=[
            pl.BlockSpec((1, 512, K * C), lambda b_, i: (b_, i, 0)),
            pl.BlockSpec((1, 512, C), lambda b_, i: (b_, i, 0)),
            pl.BlockSpec((C, C), lambda b_, i: (0, 0)),
            pl.BlockSpec((1, C), lambda b_, i: (0, 0)),
            pl.BlockSpec((1, C), lambda b_, i: (0, 0)),
            pl.BlockSpec((1, C), lambda b_, i: (0, 0)),
            pl.BlockSpec((1, 512, C), lambda b_, i: (b_, i, 0)),
        ],
        out_specs=pl.BlockSpec((1, 512, C), lambda b_, i: (b_, i, 0)),
        out_shape=jax.ShapeDtypeStruct((B, N2, C), jnp.float32),
    )(g2, c2, w21, b21, lng, lnb, res2)


def _topk_idx(d_t, n_cand, n_ctr, row_off):
    d = jnp.transpose(d_t, (0, 2, 1))
    _, idx = lax.top_k(-d, K)
    off = (jnp.arange(B, dtype=jnp.int32) * row_off)[:, None, None]
    return (idx.astype(jnp.int32) + off).reshape(-1)


def kernel(position_matrix, channel_matrix, n_select_0, n_select_1, n_select_2, W1_0, b1_0, W1_1, b1_1, Wres1, bres1, ln1_g, ln1_b, W2_0, b2_0, W2_1, b2_1, Wres2, bres2, ln2_g, ln2_b):
    pos = position_matrix
    feat = channel_matrix

    pos8 = jnp.pad(pos, ((0, 0), (0, 0), (0, 5)))
    npos1T = jnp.transpose(pos8[:, :N1], (0, 2, 1))
    pos2_8 = pos8[:, :N1]
    npos2T = jnp.transpose(pos8[:, :N2], (0, 2, 1))
    f_pad = jnp.concatenate(
        [feat, pos, jnp.zeros((B, N0, 5), jnp.float32)], axis=-1)
    w10p = jnp.pad(W1_0, ((0, 5), (0, 0)))
    w1p8 = jnp.pad(W1_0[C:], ((0, 5), (0, 0)))
    w2p8 = jnp.pad(W2_0[C:], ((0, 5), (0, 0)))
    w2f = W2_0[:C]
    r1 = lambda v: v.reshape(1, C)
    feat1 = feat[:, :N1]

    d1_t, z1 = _call_d1(pos8, npos1T, f_pad, w10p)
    d2_t, c1, res1, np2, c2 = _call_d2(
        pos2_8, npos2T, feat1, Wres1, r1(bres1), w1p8, r1(b1_0), w2p8, r1(b2_0))

    idx1 = _topk_idx(d1_t, N0, N1, N0)
    g1 = jnp.take(z1.reshape(B * N0, C), idx1, axis=0).reshape(B, N1, K * C)

    och = _call_mlp(g1, c1, W1_1, r1(b1_1), r1(ln1_g), r1(ln1_b), N1, 512)

    z2, res2 = _call_b2(och, np2, w2f, Wres2, r1(bres2), res1)

    idx2 = _topk_idx(d2_t, N1, N2, N1)
    g2 = jnp.take(z2.reshape(B * N1, C), idx2, axis=0).reshape(B, N2, K * C)

    out_ch = _call_c(g2, c2, W2_1, r1(b2_1), r1(ln2_g), r1(ln2_b), res2)
    return (pos[:, :N2], out_ch)
